# baseline (speedup 1.0000x reference)
import dataclasses
import functools

import jax
import jax.numpy as jnp
from jax import lax
from jax.experimental import pallas as pl
from jax.experimental.pallas import tpu as pltpu
from jax.experimental.pallas import tpu_sc as plsc

NUM_CORES = 2
NUM_SUBCORES = 16
LANES = 16
NW = NUM_CORES * NUM_SUBCORES

BATCH = 16384
FEATURES = 32
BPW = BATCH // NW
GATHER_WINDOW = 128
NWIN = BPW // GATHER_WINDOW
CHUNKS = BPW // LANES


def _mesh():
    return plsc.VectorSubcoreMesh(
        core_axis_name="c",
        subcore_axis_name="s",
        num_cores=NUM_CORES,
        num_subcores=NUM_SUBCORES,
    )


def _compiler_params():
    cp = pltpu.CompilerParams()
    if "needs_layout_passes" in pltpu.CompilerParams.__dataclass_fields__:
        cp = dataclasses.replace(cp, needs_layout_passes=False)
    if "use_tc_tiling_on_sc" in pltpu.CompilerParams.__dataclass_fields__:
        cp = dataclasses.replace(cp, use_tc_tiling_on_sc=False)
    return cp


@functools.partial(
    pl.kernel,
    out_type=jax.ShapeDtypeStruct((BATCH,), jnp.float32),
    mesh=_mesh(),
    compiler_params=_compiler_params(),
    scratch_types=[
        pltpu.VMEM((NWIN, GATHER_WINDOW), jnp.int32),
        pltpu.VMEM((NWIN, GATHER_WINDOW), jnp.int32),
        pltpu.VMEM((BPW, FEATURES), jnp.float32),
        pltpu.VMEM((BPW, FEATURES), jnp.float32),
        pltpu.VMEM((FEATURES, LANES), jnp.float32),
        pltpu.VMEM((BPW,), jnp.float32),
        pltpu.SemaphoreType.DMA,
        pltpu.SemaphoreType.DMA,
    ],
)
def _mf_sc(user_hbm, item_hbm, uemb_hbm, iemb_hbm, w_hbm, out_hbm,
           uidx_v, iidx_v, urows_v, irows_v, w_v, out_v, sem_u, sem_i):
    wid = lax.axis_index("s") * NUM_CORES + lax.axis_index("c")
    base = wid * BPW

    pltpu.sync_copy(user_hbm.at[wid], uidx_v)
    pltpu.sync_copy(item_hbm.at[wid], iidx_v)
    pltpu.sync_copy(w_hbm, w_v)

    copies = []
    for j in range(NWIN):
        copies.append(pltpu.async_copy(
            uemb_hbm.at[uidx_v.at[j]],
            urows_v.at[pl.ds(j * GATHER_WINDOW, GATHER_WINDOW)],
            sem_u,
        ))
        copies.append(pltpu.async_copy(
            iemb_hbm.at[iidx_v.at[j]],
            irows_v.at[pl.ds(j * GATHER_WINDOW, GATHER_WINDOW)],
            sem_i,
        ))
    for c in copies:
        c.wait()

    lane = lax.iota(jnp.int32, LANES)

    @pl.loop(0, CHUNKS)
    def _(c):
        r_ids = c * LANES + lane
        acc = jnp.zeros((LANES,), jnp.float32)
        for f in range(FEATURES):
            f_vec = jnp.full((LANES,), f, jnp.int32)
            u = plsc.load_gather(urows_v, [r_ids, f_vec])
            iv = plsc.load_gather(irows_v, [r_ids, f_vec])
            acc = acc + u * iv * w_v[f, :]
        out_v[pl.ds(c * LANES, LANES)] = acc

    pltpu.sync_copy(out_v, out_hbm.at[pl.ds(base, BPW)])


def kernel(user, item, user_emb, item_emb, W):
    user_w = user.astype(jnp.int32).reshape(NW, NWIN, GATHER_WINDOW)
    item_w = item.astype(jnp.int32).reshape(NW, NWIN, GATHER_WINDOW)
    w_b = jnp.broadcast_to(W.reshape(FEATURES, 1), (FEATURES, LANES))
    return _mf_sc(user_w, item_w, user_emb, item_emb, w_b)
